# Initial kernel scaffold; baseline (speedup 1.0000x reference)
import jax
import jax.numpy as jnp
from jax.experimental import pallas as pl


def kernel(x, edge_index, batch, W1, b1, W2, b2, W3, b3, s1, s2, s3, Wl1, bl1, Wl2, bl2, Wl3, bl3):
    raise NotImplementedError("write your pallas kernel here")


# speedup vs baseline: 7.1655x; 7.1655x over previous
import functools

import jax
import jax.numpy as jnp
from jax import lax
from jax.experimental import pallas as pl
from jax.experimental.pallas import tpu as pltpu
from jax.experimental.pallas import tpu_sc as plsc

N = 10000
E = 320000
H = 128
B = 64
C = 10

NC = 2
NS = 16
NW = NC * NS
EPW = E // NW
CH = 80
NCHUNK = EPW // CH
RPT = 624
TAIL = N - NS * RPT
TAIL_OFF = NS * RPT


def _sc_degree(dst, ones_ch, zeros_n):
    mesh = plsc.VectorSubcoreMesh(core_axis_name="c", subcore_axis_name="s")

    @functools.partial(
        pl.kernel,
        out_type=jax.ShapeDtypeStruct((NC * N,), jnp.float32),
        mesh=mesh,
    scratch_types=[
            pltpu.VMEM((CH,), jnp.int32),
            pltpu.VMEM((CH,), jnp.float32),
            pltpu.VMEM((RPT + TAIL,), jnp.float32),
            pltpu.VMEM_SHARED((N,), jnp.float32),
        ],
    )
    def k(dst_hbm, ones_hbm, zero_hbm, out_hbm, idx_v, ones_v, stage_v, acc_sh):
        cid = lax.axis_index("c")
        sid = lax.axis_index("s")
        wid = sid * NC + cid

        pltpu.sync_copy(ones_hbm, ones_v)
        pltpu.sync_copy(zero_hbm.at[pl.ds(0, RPT + TAIL)], stage_v)
        pltpu.sync_copy(
            stage_v.at[pl.ds(0, RPT)], acc_sh.at[pl.ds(sid * RPT, RPT)]
        )

        @pl.when(sid == NS - 1)
        def _():
            pltpu.sync_copy(
                stage_v.at[pl.ds(RPT, TAIL)], acc_sh.at[pl.ds(TAIL_OFF, TAIL)]
            )

        plsc.subcore_barrier()

        def chunk(i, carry):
            base = wid * EPW + i * CH
            pltpu.sync_copy(dst_hbm.at[pl.ds(base, CH)], idx_v)
            pltpu.sync_copy(ones_v, acc_sh.at[idx_v], add=True)
            return carry

        lax.fori_loop(0, NCHUNK, chunk, 0)
        plsc.subcore_barrier()
        pltpu.sync_copy(acc_sh.at[pl.ds(sid * RPT, RPT)], stage_v.at[pl.ds(0, RPT)])

        @pl.when(sid == NS - 1)
        def _():
            pltpu.sync_copy(
                acc_sh.at[pl.ds(TAIL_OFF, TAIL)], stage_v.at[pl.ds(RPT, TAIL)]
            )

        pltpu.sync_copy(
            stage_v.at[pl.ds(0, RPT)], out_hbm.at[pl.ds(cid * N + sid * RPT, RPT)]
        )

        @pl.when(sid == NS - 1)
        def _():
            pltpu.sync_copy(
                stage_v.at[pl.ds(RPT, TAIL)],
                out_hbm.at[pl.ds(cid * N + TAIL_OFF, TAIL)],
            )

    return k(dst, ones_ch, zeros_n)


def _sc_spmm(y, src, dst, zeros_nh):
    mesh = plsc.VectorSubcoreMesh(core_axis_name="c", subcore_axis_name="s")

    @functools.partial(
        pl.kernel,
        out_type=jax.ShapeDtypeStruct((NC, N, H), jnp.float32),
        mesh=mesh,
        scratch_types=[
            pltpu.VMEM((CH,), jnp.int32),
            pltpu.VMEM((CH,), jnp.int32),
            pltpu.VMEM((CH, H), jnp.float32),
            pltpu.VMEM_SHARED((N, H), jnp.float32),
            pltpu.SemaphoreType.DMA,
        ],
    )
    def k(y_hbm, src_hbm, dst_hbm, zero_hbm, out_hbm, si_v, di_v, rows_v, acc_sh, sem):
        cid = lax.axis_index("c")
        sid = lax.axis_index("s")
        wid = sid * NC + cid

        pltpu.sync_copy(
            zero_hbm.at[pl.ds(sid * RPT, RPT)], acc_sh.at[pl.ds(sid * RPT, RPT)]
        )

        @pl.when(sid == NS - 1)
        def _():
            pltpu.sync_copy(
                zero_hbm.at[pl.ds(TAIL_OFF, TAIL)], acc_sh.at[pl.ds(TAIL_OFF, TAIL)]
            )

        plsc.subcore_barrier()

        def chunk(i, carry):
            base = wid * EPW + i * CH
            pltpu.sync_copy(src_hbm.at[pl.ds(base, CH)], si_v)
            pltpu.sync_copy(dst_hbm.at[pl.ds(base, CH)], di_v)
            pltpu.async_copy(y_hbm.at[si_v], rows_v, sem).wait()
            pltpu.sync_copy(rows_v, acc_sh.at[di_v], add=True)
            return carry

        lax.fori_loop(0, NCHUNK, chunk, 0)
        plsc.subcore_barrier()
        pltpu.sync_copy(
            acc_sh.at[pl.ds(sid * RPT, RPT)], out_hbm.at[cid, pl.ds(sid * RPT, RPT)]
        )

        @pl.when(sid == NS - 1)
        def _():
            pltpu.sync_copy(
                acc_sh.at[pl.ds(TAIL_OFF, TAIL)],
                out_hbm.at[cid, pl.ds(TAIL_OFF, TAIL)],
            )

    return k(y, src, dst, zeros_nh)


def _tc_pre(x, W1, degp):
    def body(x_ref, w_ref, degp_ref, y_ref, inv_ref):
        deg = degp_ref[0] + degp_ref[1]
        inv = lax.rsqrt(jnp.maximum(deg, 1.0))
        h = jnp.dot(x_ref[...], w_ref[...], preferred_element_type=jnp.float32)
        y_ref[...] = h * inv
        inv_ref[...] = inv

    return pl.pallas_call(
        body,
        out_shape=(
            jax.ShapeDtypeStruct((N, H), jnp.float32),
            jax.ShapeDtypeStruct((N, 1), jnp.float32),
        ),
    )(x, W1, degp)


def _post_layer(p0, p1, inv, bvec, svec, bat_col, bat_row):
    agg = (p0 + p1) * inv + bvec
    h = jnp.maximum(agg, 0.0)
    logit = jnp.dot(h, svec, preferred_element_type=jnp.float32)
    pgate = 1.0 / (1.0 + jnp.exp(-logit))
    pc = jnp.clip(pgate, 1e-6, 1.0 - 1e-6)
    kl = jnp.mean(pc * jnp.log(2.0 * pc) + (1.0 - pc) * jnp.log(2.0 * (1.0 - pc)))
    ref_ = jnp.mean((pc - 0.5) ** 2)
    hp = h * pgate

    iota = lax.broadcasted_iota(jnp.int32, (B, N), 0)
    m = (iota == bat_row).astype(jnp.float32)
    cnt = jnp.sum(m, axis=1, keepdims=True)
    gap = jnp.dot(m, hp, preferred_element_type=jnp.float32) / jnp.maximum(cnt, 1.0)

    rows = []
    for bidx in range(B):
        mcol = bat_col == bidx
        masked = jnp.where(mcol, hp, -3.4e38)
        rows.append(jnp.max(masked, axis=0, keepdims=True))
    gmax = jnp.concatenate(rows, axis=0)
    gmax = jnp.where(cnt > 0.0, gmax, 0.0)
    return hp, kl, ref_, jnp.concatenate([gmax, gap], axis=1)


def _tc_mid(p, inv, bvec, svec, Wn, bat_col, bat_row):
    def body(p_ref, inv_ref, b_ref, s_ref, w_ref, bc_ref, br_ref,
             y_ref, x1_ref, kl_ref, ref_ref):
        hp, kl, ref_, xout = _post_layer(
            p_ref[0], p_ref[1], inv_ref[...], b_ref[...], s_ref[...],
            bc_ref[...], br_ref[...])
        x1_ref[...] = xout
        kl_ref[...] = kl.reshape(1, 1)
        ref_ref[...] = ref_.reshape(1, 1)
        y_ref[...] = (
            jnp.dot(hp, w_ref[...], preferred_element_type=jnp.float32)
            * inv_ref[...]
        )

    return pl.pallas_call(
        body,
        out_shape=(
            jax.ShapeDtypeStruct((N, H), jnp.float32),
            jax.ShapeDtypeStruct((B, 2 * H), jnp.float32),
            jax.ShapeDtypeStruct((1, 1), jnp.float32),
            jax.ShapeDtypeStruct((1, 1), jnp.float32),
        ),
    )(p, inv, bvec, svec, Wn, bat_col, bat_row)


def _tc_final(p, inv, bvec, svec, bat_col, bat_row, x1, x2, kl1, kl2, ref1, ref2,
              Wl1, bl1, Wl2, bl2, Wl3, bl3):
    def body(p_ref, inv_ref, b_ref, s_ref, bc_ref, br_ref, x1_ref, x2_ref,
             kl1_ref, kl2_ref, ref1_ref, ref2_ref,
             wl1_ref, bl1_ref, wl2_ref, bl2_ref, wl3_ref, bl3_ref,
             logits_ref, kl_ref, ref_ref):
        hp, kl3, ref3, x3 = _post_layer(
            p_ref[0], p_ref[1], inv_ref[...], b_ref[...], s_ref[...],
            bc_ref[...], br_ref[...])
        del hp
        kl_ref[...] = (kl1_ref[...] + kl2_ref[...] + kl3.reshape(1, 1))
        ref_ref[...] = (ref1_ref[...] + ref2_ref[...] + ref3.reshape(1, 1))

        g = (jnp.maximum(x1_ref[...], 0.0)
             + jnp.maximum(x2_ref[...], 0.0)
             + jnp.maximum(x3, 0.0))
        g = jnp.maximum(
            jnp.dot(g, wl1_ref[...], preferred_element_type=jnp.float32)
            + bl1_ref[...], 0.0)
        g = jnp.maximum(
            jnp.dot(g, wl2_ref[...], preferred_element_type=jnp.float32)
            + bl2_ref[...], 0.0)
        z = (jnp.dot(g, wl3_ref[...], preferred_element_type=jnp.float32)
             + bl3_ref[...])
        zm = jnp.max(z, axis=1, keepdims=True)
        ze = z - zm
        logits_ref[...] = ze - jnp.log(jnp.sum(jnp.exp(ze), axis=1, keepdims=True))

    return pl.pallas_call(
        body,
        out_shape=(
            jax.ShapeDtypeStruct((B, C), jnp.float32),
            jax.ShapeDtypeStruct((1, 1), jnp.float32),
            jax.ShapeDtypeStruct((1, 1), jnp.float32),
        ),
    )(p, inv, bvec, svec, bat_col, bat_row, x1, x2, kl1, kl2, ref1, ref2,
      Wl1, bl1, Wl2, bl2, Wl3, bl3)


def kernel(x, edge_index, batch, W1, b1, W2, b2, W3, b3, s1, s2, s3,
           Wl1, bl1, Wl2, bl2, Wl3, bl3):
    src = edge_index[0]
    dst = edge_index[1]
    bat_col = batch.reshape(N, 1)
    bat_row = batch.reshape(1, N)
    zeros_nh = jnp.zeros((N, H), jnp.float32)
    zeros_n = jnp.zeros((N,), jnp.float32)
    ones_ch = jnp.ones((CH,), jnp.float32)

    degp = _sc_degree(dst, ones_ch, zeros_n).reshape(NC, N, 1)
    y1, inv = _tc_pre(x, W1, degp)
    p1 = _sc_spmm(y1, src, dst, zeros_nh)
    y2, x1, kl1, ref1 = _tc_mid(
        p1, inv, b1.reshape(1, H), s1.reshape(H, 1), W2, bat_col, bat_row)
    p2 = _sc_spmm(y2, src, dst, zeros_nh)
    y3, x2, kl2, ref2 = _tc_mid(
        p2, inv, b2.reshape(1, H), s2.reshape(H, 1), W3, bat_col, bat_row)
    p3 = _sc_spmm(y3, src, dst, zeros_nh)
    logits, kl, refo = _tc_final(
        p3, inv, b3.reshape(1, H), s3.reshape(H, 1), bat_col, bat_row,
        x1, x2, kl1, kl2, ref1, ref2,
        Wl1, bl1.reshape(1, H), Wl2, bl2.reshape(1, H // 2),
        Wl3, bl3.reshape(1, C))
    return (logits, kl.reshape(()), refo.reshape(()))


# speedup vs baseline: 9.9457x; 1.3880x over previous
import functools

import jax
import jax.numpy as jnp
from jax import lax
from jax.experimental import pallas as pl
from jax.experimental.pallas import tpu as pltpu
from jax.experimental.pallas import tpu_sc as plsc

N = 10000
E = 320000
H = 128
B = 64
C = 10

NC = 2
NS = 16
NW = NC * NS
EPW = E // NW
CH = 80
NCHUNK = EPW // CH
RPT = 624
TAIL = N - NS * RPT
TAIL_OFF = NS * RPT


def _sc_degree(dst, ones_ch, zeros_n):
    mesh = plsc.VectorSubcoreMesh(core_axis_name="c", subcore_axis_name="s")

    @functools.partial(
        pl.kernel,
        out_type=jax.ShapeDtypeStruct((NC * N,), jnp.float32),
        mesh=mesh,
    scratch_types=[
            pltpu.VMEM((CH,), jnp.int32),
            pltpu.VMEM((CH,), jnp.int32),
            pltpu.VMEM((CH,), jnp.float32),
            pltpu.VMEM((RPT + TAIL,), jnp.float32),
            pltpu.VMEM_SHARED((N,), jnp.float32),
            pltpu.SemaphoreType.DMA,
        ],
    )
    def k(dst_hbm, ones_hbm, zero_hbm, out_hbm, idx_v, idx2_v, ones_v, stage_v,
          acc_sh, isem):
        cid = lax.axis_index("c")
        sid = lax.axis_index("s")
        wid = sid * NC + cid

        pltpu.sync_copy(ones_hbm, ones_v)
        pltpu.sync_copy(zero_hbm.at[pl.ds(0, RPT + TAIL)], stage_v)
        pltpu.sync_copy(
            stage_v.at[pl.ds(0, RPT)], acc_sh.at[pl.ds(sid * RPT, RPT)]
        )

        @pl.when(sid == NS - 1)
        def _():
            pltpu.sync_copy(
                stage_v.at[pl.ds(RPT, TAIL)], acc_sh.at[pl.ds(TAIL_OFF, TAIL)]
            )

        plsc.subcore_barrier()

        pltpu.sync_copy(dst_hbm.at[pl.ds(wid * EPW, CH)], idx_v)

        def pair(j, carry):
            b = wid * EPW + 2 * j * CH
            ld1 = pltpu.async_copy(dst_hbm.at[pl.ds(b + CH, CH)], idx2_v, isem)
            pltpu.sync_copy(ones_v, acc_sh.at[idx_v], add=True)
            ld1.wait()
            ld2 = pltpu.async_copy(dst_hbm.at[pl.ds(b + 2 * CH, CH)], idx_v, isem)
            pltpu.sync_copy(ones_v, acc_sh.at[idx2_v], add=True)
            ld2.wait()
            return carry

        lax.fori_loop(0, (NCHUNK - 1) // 2, pair, 0)
        pltpu.sync_copy(ones_v, acc_sh.at[idx_v], add=True)
        plsc.subcore_barrier()
        pltpu.sync_copy(acc_sh.at[pl.ds(sid * RPT, RPT)], stage_v.at[pl.ds(0, RPT)])

        @pl.when(sid == NS - 1)
        def _():
            pltpu.sync_copy(
                acc_sh.at[pl.ds(TAIL_OFF, TAIL)], stage_v.at[pl.ds(RPT, TAIL)]
            )

        pltpu.sync_copy(
            stage_v.at[pl.ds(0, RPT)], out_hbm.at[pl.ds(cid * N + sid * RPT, RPT)]
        )

        @pl.when(sid == NS - 1)
        def _():
            pltpu.sync_copy(
                stage_v.at[pl.ds(RPT, TAIL)],
                out_hbm.at[pl.ds(cid * N + TAIL_OFF, TAIL)],
            )

    return k(dst, ones_ch, zeros_n)


def _sc_spmm(y, src, dst, zeros_nh):
    mesh = plsc.VectorSubcoreMesh(core_axis_name="c", subcore_axis_name="s")

    @functools.partial(
        pl.kernel,
        out_type=jax.ShapeDtypeStruct((NC, N, H), jnp.float32),
        mesh=mesh,
        scratch_types=[
            pltpu.VMEM((CH,), jnp.int32),
            pltpu.VMEM((CH,), jnp.int32),
            pltpu.VMEM((CH,), jnp.int32),
            pltpu.VMEM((CH,), jnp.int32),
            pltpu.VMEM((CH, H), jnp.float32),
            pltpu.VMEM((CH, H), jnp.float32),
            pltpu.VMEM_SHARED((N, H), jnp.float32),
            pltpu.SemaphoreType.DMA,
        ],
    )
    def k(y_hbm, src_hbm, dst_hbm, zero_hbm, out_hbm,
          si0, di0, si1, di1, rows0, rows1, acc_sh, gsem):
        cid = lax.axis_index("c")
        sid = lax.axis_index("s")
        wid = sid * NC + cid

        pltpu.sync_copy(
            zero_hbm.at[pl.ds(sid * RPT, RPT)], acc_sh.at[pl.ds(sid * RPT, RPT)]
        )

        @pl.when(sid == NS - 1)
        def _():
            pltpu.sync_copy(
                zero_hbm.at[pl.ds(TAIL_OFF, TAIL)], acc_sh.at[pl.ds(TAIL_OFF, TAIL)]
            )

        plsc.subcore_barrier()

        base0 = wid * EPW
        pltpu.sync_copy(src_hbm.at[pl.ds(base0, CH)], si0)
        pltpu.sync_copy(dst_hbm.at[pl.ds(base0, CH)], di0)
        pltpu.async_copy(y_hbm.at[si0], rows0, gsem)
        pltpu.sync_copy(src_hbm.at[pl.ds(base0 + CH, CH)], si1)
        pltpu.sync_copy(dst_hbm.at[pl.ds(base0 + CH, CH)], di1)

        def pair(j, carry):
            b = base0 + 2 * j * CH
            pltpu.make_async_copy(y_hbm.at[si0], rows0, gsem).wait()
            pltpu.async_copy(y_hbm.at[si1], rows1, gsem)
            pltpu.sync_copy(rows0, acc_sh.at[di0], add=True)
            pltpu.sync_copy(src_hbm.at[pl.ds(b + 2 * CH, CH)], si0)
            pltpu.sync_copy(dst_hbm.at[pl.ds(b + 2 * CH, CH)], di0)
            pltpu.make_async_copy(y_hbm.at[si1], rows1, gsem).wait()
            pltpu.async_copy(y_hbm.at[si0], rows0, gsem)
            pltpu.sync_copy(rows1, acc_sh.at[di1], add=True)
            b3 = jnp.minimum(b + 3 * CH, E - CH)
            pltpu.sync_copy(src_hbm.at[pl.ds(b3, CH)], si1)
            pltpu.sync_copy(dst_hbm.at[pl.ds(b3, CH)], di1)
            return carry

        lax.fori_loop(0, (NCHUNK - 1) // 2, pair, 0)
        pltpu.make_async_copy(y_hbm.at[si0], rows0, gsem).wait()
        pltpu.sync_copy(rows0, acc_sh.at[di0], add=True)
        plsc.subcore_barrier()
        pltpu.sync_copy(
            acc_sh.at[pl.ds(sid * RPT, RPT)], out_hbm.at[cid, pl.ds(sid * RPT, RPT)]
        )

        @pl.when(sid == NS - 1)
        def _():
            pltpu.sync_copy(
                acc_sh.at[pl.ds(TAIL_OFF, TAIL)],
                out_hbm.at[cid, pl.ds(TAIL_OFF, TAIL)],
            )

    return k(y, src, dst, zeros_nh)


def _tc_pre(x, W1, degp):
    def body(x_ref, w_ref, degp_ref, y_ref, inv_ref):
        deg = degp_ref[0] + degp_ref[1]
        inv = lax.rsqrt(jnp.maximum(deg, 1.0))
        h = jnp.dot(x_ref[...], w_ref[...], preferred_element_type=jnp.float32)
        y_ref[...] = h * inv
        inv_ref[...] = inv

    return pl.pallas_call(
        body,
        out_shape=(
            jax.ShapeDtypeStruct((N, H), jnp.float32),
            jax.ShapeDtypeStruct((N, 1), jnp.float32),
        ),
    )(x, W1, degp)


def _post_layer(p0, p1, inv, bvec, svec, bat_col, bat_row):
    agg = (p0 + p1) * inv + bvec
    h = jnp.maximum(agg, 0.0)
    logit = jnp.dot(h, svec, preferred_element_type=jnp.float32)
    pgate = 1.0 / (1.0 + jnp.exp(-logit))
    pc = jnp.clip(pgate, 1e-6, 1.0 - 1e-6)
    kl = jnp.mean(pc * jnp.log(2.0 * pc) + (1.0 - pc) * jnp.log(2.0 * (1.0 - pc)))
    ref_ = jnp.mean((pc - 0.5) ** 2)
    hp = h * pgate

    iota = lax.broadcasted_iota(jnp.int32, (B, N), 0)
    m = (iota == bat_row).astype(jnp.float32)
    cnt = jnp.sum(m, axis=1, keepdims=True)
    gap = jnp.dot(m, hp, preferred_element_type=jnp.float32) / jnp.maximum(cnt, 1.0)

    rows = []
    for bidx in range(B):
        mcol = bat_col == bidx
        masked = jnp.where(mcol, hp, -3.4e38)
        rows.append(jnp.max(masked, axis=0, keepdims=True))
    gmax = jnp.concatenate(rows, axis=0)
    gmax = jnp.where(cnt > 0.0, gmax, 0.0)
    return hp, kl, ref_, jnp.concatenate([gmax, gap], axis=1)


def _tc_mid(p, inv, bvec, svec, Wn, bat_col, bat_row):
    def body(p_ref, inv_ref, b_ref, s_ref, w_ref, bc_ref, br_ref,
             y_ref, x1_ref, kl_ref, ref_ref):
        hp, kl, ref_, xout = _post_layer(
            p_ref[0], p_ref[1], inv_ref[...], b_ref[...], s_ref[...],
            bc_ref[...], br_ref[...])
        x1_ref[...] = xout
        kl_ref[...] = kl.reshape(1, 1)
        ref_ref[...] = ref_.reshape(1, 1)
        y_ref[...] = (
            jnp.dot(hp, w_ref[...], preferred_element_type=jnp.float32)
            * inv_ref[...]
        )

    return pl.pallas_call(
        body,
        out_shape=(
            jax.ShapeDtypeStruct((N, H), jnp.float32),
            jax.ShapeDtypeStruct((B, 2 * H), jnp.float32),
            jax.ShapeDtypeStruct((1, 1), jnp.float32),
            jax.ShapeDtypeStruct((1, 1), jnp.float32),
        ),
    )(p, inv, bvec, svec, Wn, bat_col, bat_row)


def _tc_final(p, inv, bvec, svec, bat_col, bat_row, x1, x2, kl1, kl2, ref1, ref2,
              Wl1, bl1, Wl2, bl2, Wl3, bl3):
    def body(p_ref, inv_ref, b_ref, s_ref, bc_ref, br_ref, x1_ref, x2_ref,
             kl1_ref, kl2_ref, ref1_ref, ref2_ref,
             wl1_ref, bl1_ref, wl2_ref, bl2_ref, wl3_ref, bl3_ref,
             logits_ref, kl_ref, ref_ref):
        hp, kl3, ref3, x3 = _post_layer(
            p_ref[0], p_ref[1], inv_ref[...], b_ref[...], s_ref[...],
            bc_ref[...], br_ref[...])
        del hp
        kl_ref[...] = (kl1_ref[...] + kl2_ref[...] + kl3.reshape(1, 1))
        ref_ref[...] = (ref1_ref[...] + ref2_ref[...] + ref3.reshape(1, 1))

        g = (jnp.maximum(x1_ref[...], 0.0)
             + jnp.maximum(x2_ref[...], 0.0)
             + jnp.maximum(x3, 0.0))
        g = jnp.maximum(
            jnp.dot(g, wl1_ref[...], preferred_element_type=jnp.float32)
            + bl1_ref[...], 0.0)
        g = jnp.maximum(
            jnp.dot(g, wl2_ref[...], preferred_element_type=jnp.float32)
            + bl2_ref[...], 0.0)
        z = (jnp.dot(g, wl3_ref[...], preferred_element_type=jnp.float32)
             + bl3_ref[...])
        zm = jnp.max(z, axis=1, keepdims=True)
        ze = z - zm
        logits_ref[...] = ze - jnp.log(jnp.sum(jnp.exp(ze), axis=1, keepdims=True))

    return pl.pallas_call(
        body,
        out_shape=(
            jax.ShapeDtypeStruct((B, C), jnp.float32),
            jax.ShapeDtypeStruct((1, 1), jnp.float32),
            jax.ShapeDtypeStruct((1, 1), jnp.float32),
        ),
    )(p, inv, bvec, svec, bat_col, bat_row, x1, x2, kl1, kl2, ref1, ref2,
      Wl1, bl1, Wl2, bl2, Wl3, bl3)


def kernel(x, edge_index, batch, W1, b1, W2, b2, W3, b3, s1, s2, s3,
           Wl1, bl1, Wl2, bl2, Wl3, bl3):
    src = edge_index[0]
    dst = edge_index[1]
    bat_col = batch.reshape(N, 1)
    bat_row = batch.reshape(1, N)
    zeros_nh = jnp.zeros((N, H), jnp.float32)
    zeros_n = jnp.zeros((N,), jnp.float32)
    ones_ch = jnp.ones((CH,), jnp.float32)

    degp = _sc_degree(dst, ones_ch, zeros_n).reshape(NC, N, 1)
    y1, inv = _tc_pre(x, W1, degp)
    p1 = _sc_spmm(y1, src, dst, zeros_nh)
    y2, x1, kl1, ref1 = _tc_mid(
        p1, inv, b1.reshape(1, H), s1.reshape(H, 1), W2, bat_col, bat_row)
    p2 = _sc_spmm(y2, src, dst, zeros_nh)
    y3, x2, kl2, ref2 = _tc_mid(
        p2, inv, b2.reshape(1, H), s2.reshape(H, 1), W3, bat_col, bat_row)
    p3 = _sc_spmm(y3, src, dst, zeros_nh)
    logits, kl, refo = _tc_final(
        p3, inv, b3.reshape(1, H), s3.reshape(H, 1), bat_col, bat_row,
        x1, x2, kl1, kl2, ref1, ref2,
        Wl1, bl1.reshape(1, H), Wl2, bl2.reshape(1, H // 2),
        Wl3, bl3.reshape(1, C))
    return (logits, kl.reshape(()), refo.reshape(()))


# speedup vs baseline: 12.7541x; 1.2824x over previous
import functools

import jax
import jax.numpy as jnp
from jax import lax
from jax.experimental import pallas as pl
from jax.experimental.pallas import tpu as pltpu
from jax.experimental.pallas import tpu_sc as plsc

N = 10000
E = 320000
H = 128
B = 64
C = 10

NC = 2
NS = 16
NW = NC * NS
EPW = E // NW
CH = 80
NCHUNK = EPW // CH
RPT = 624
TAIL = N - NS * RPT
TAIL_OFF = NS * RPT


def _sc_degree(dst, ones_ch, zeros_n):
    mesh = plsc.VectorSubcoreMesh(core_axis_name="c", subcore_axis_name="s")

    @functools.partial(
        pl.kernel,
        out_type=jax.ShapeDtypeStruct((NC * N,), jnp.float32),
        mesh=mesh,
    scratch_types=[
            pltpu.VMEM((CH,), jnp.int32),
            pltpu.VMEM((CH,), jnp.int32),
            pltpu.VMEM((CH,), jnp.float32),
            pltpu.VMEM((RPT + TAIL,), jnp.float32),
            pltpu.VMEM_SHARED((N,), jnp.float32),
            pltpu.SemaphoreType.DMA,
        ],
    )
    def k(dst_hbm, ones_hbm, zero_hbm, out_hbm, idx_v, idx2_v, ones_v, stage_v,
          acc_sh, isem):
        cid = lax.axis_index("c")
        sid = lax.axis_index("s")
        wid = sid * NC + cid

        pltpu.sync_copy(ones_hbm, ones_v)
        pltpu.sync_copy(zero_hbm.at[pl.ds(0, RPT + TAIL)], stage_v)
        pltpu.sync_copy(
            stage_v.at[pl.ds(0, RPT)], acc_sh.at[pl.ds(sid * RPT, RPT)]
        )

        @pl.when(sid == NS - 1)
        def _():
            pltpu.sync_copy(
                stage_v.at[pl.ds(RPT, TAIL)], acc_sh.at[pl.ds(TAIL_OFF, TAIL)]
            )

        plsc.subcore_barrier()

        pltpu.sync_copy(dst_hbm.at[pl.ds(wid * EPW, CH)], idx_v)

        def pair(j, carry):
            b = wid * EPW + 2 * j * CH
            ld1 = pltpu.async_copy(dst_hbm.at[pl.ds(b + CH, CH)], idx2_v, isem)
            pltpu.sync_copy(ones_v, acc_sh.at[idx_v], add=True)
            ld1.wait()
            ld2 = pltpu.async_copy(dst_hbm.at[pl.ds(b + 2 * CH, CH)], idx_v, isem)
            pltpu.sync_copy(ones_v, acc_sh.at[idx2_v], add=True)
            ld2.wait()
            return carry

        lax.fori_loop(0, (NCHUNK - 1) // 2, pair, 0)
        pltpu.sync_copy(ones_v, acc_sh.at[idx_v], add=True)
        plsc.subcore_barrier()
        pltpu.sync_copy(acc_sh.at[pl.ds(sid * RPT, RPT)], stage_v.at[pl.ds(0, RPT)])

        @pl.when(sid == NS - 1)
        def _():
            pltpu.sync_copy(
                acc_sh.at[pl.ds(TAIL_OFF, TAIL)], stage_v.at[pl.ds(RPT, TAIL)]
            )

        pltpu.sync_copy(
            stage_v.at[pl.ds(0, RPT)], out_hbm.at[pl.ds(cid * N + sid * RPT, RPT)]
        )

        @pl.when(sid == NS - 1)
        def _():
            pltpu.sync_copy(
                stage_v.at[pl.ds(RPT, TAIL)],
                out_hbm.at[pl.ds(cid * N + TAIL_OFF, TAIL)],
            )

    return k(dst, ones_ch, zeros_n)


def _sc_spmm(y, src, dst, zeros_nh):
    mesh = plsc.VectorSubcoreMesh(core_axis_name="c", subcore_axis_name="s")

    @functools.partial(
        pl.kernel,
        out_type=jax.ShapeDtypeStruct((NC, N, H), jnp.float32),
        mesh=mesh,
        scratch_types=[
            pltpu.VMEM((CH,), jnp.int32),
            pltpu.VMEM((CH,), jnp.int32),
            pltpu.VMEM((CH,), jnp.int32),
            pltpu.VMEM((CH,), jnp.int32),
            pltpu.VMEM((CH, H), jnp.float32),
            pltpu.VMEM((CH, H), jnp.float32),
            pltpu.VMEM_SHARED((N, H), jnp.float32),
            pltpu.SemaphoreType.DMA,
        ],
    )
    def k(y_hbm, src_hbm, dst_hbm, zero_hbm, out_hbm,
          si0, di0, si1, di1, rows0, rows1, acc_sh, gsem):
        cid = lax.axis_index("c")
        sid = lax.axis_index("s")
        wid = sid * NC + cid

        pltpu.sync_copy(
            zero_hbm.at[pl.ds(sid * RPT, RPT)], acc_sh.at[pl.ds(sid * RPT, RPT)]
        )

        @pl.when(sid == NS - 1)
        def _():
            pltpu.sync_copy(
                zero_hbm.at[pl.ds(TAIL_OFF, TAIL)], acc_sh.at[pl.ds(TAIL_OFF, TAIL)]
            )

        plsc.subcore_barrier()

        base0 = wid * EPW
        pltpu.sync_copy(src_hbm.at[pl.ds(base0, CH)], si0)
        pltpu.sync_copy(dst_hbm.at[pl.ds(base0, CH)], di0)
        pltpu.async_copy(y_hbm.at[si0], rows0, gsem)
        pltpu.sync_copy(src_hbm.at[pl.ds(base0 + CH, CH)], si1)
        pltpu.sync_copy(dst_hbm.at[pl.ds(base0 + CH, CH)], di1)

        def pair(j, carry):
            b = base0 + 2 * j * CH
            pltpu.make_async_copy(y_hbm.at[si0], rows0, gsem).wait()
            pltpu.async_copy(y_hbm.at[si1], rows1, gsem)
            pltpu.sync_copy(rows0, acc_sh.at[di0], add=True)
            pltpu.sync_copy(src_hbm.at[pl.ds(b + 2 * CH, CH)], si0)
            pltpu.sync_copy(dst_hbm.at[pl.ds(b + 2 * CH, CH)], di0)
            pltpu.make_async_copy(y_hbm.at[si1], rows1, gsem).wait()
            pltpu.async_copy(y_hbm.at[si0], rows0, gsem)
            pltpu.sync_copy(rows1, acc_sh.at[di1], add=True)
            b3 = jnp.minimum(b + 3 * CH, E - CH)
            pltpu.sync_copy(src_hbm.at[pl.ds(b3, CH)], si1)
            pltpu.sync_copy(dst_hbm.at[pl.ds(b3, CH)], di1)
            return carry

        lax.fori_loop(0, (NCHUNK - 1) // 2, pair, 0)
        pltpu.make_async_copy(y_hbm.at[si0], rows0, gsem).wait()
        pltpu.sync_copy(rows0, acc_sh.at[di0], add=True)
        plsc.subcore_barrier()
        pltpu.sync_copy(
            acc_sh.at[pl.ds(sid * RPT, RPT)], out_hbm.at[cid, pl.ds(sid * RPT, RPT)]
        )

        @pl.when(sid == NS - 1)
        def _():
            pltpu.sync_copy(
                acc_sh.at[pl.ds(TAIL_OFF, TAIL)],
                out_hbm.at[cid, pl.ds(TAIL_OFF, TAIL)],
            )

    return k(y, src, dst, zeros_nh)


def _tc_pre(x, W1, degp):
    def body(x_ref, w_ref, degp_ref, y_ref, inv_ref):
        deg = degp_ref[0] + degp_ref[1]
        inv = lax.rsqrt(jnp.maximum(deg, 1.0))
        h = jnp.dot(x_ref[...], w_ref[...], preferred_element_type=jnp.float32)
        y_ref[...] = h * inv
        inv_ref[...] = inv

    return pl.pallas_call(
        body,
        out_shape=(
            jax.ShapeDtypeStruct((N, H), jnp.float32),
            jax.ShapeDtypeStruct((N, 1), jnp.float32),
        ),
    )(x, W1, degp)


def _post_layer(p0, p1, inv, bvec, svec, bat_col, bat_row):
    agg = (p0 + p1) * inv + bvec
    h = jnp.maximum(agg, 0.0)
    logit = jnp.dot(h, svec, preferred_element_type=jnp.float32)
    pgate = 1.0 / (1.0 + jnp.exp(-logit))
    pc = jnp.clip(pgate, 1e-6, 1.0 - 1e-6)
    kl = jnp.mean(pc * jnp.log(2.0 * pc) + (1.0 - pc) * jnp.log(2.0 * (1.0 - pc)))
    ref_ = jnp.mean((pc - 0.5) ** 2)
    hp = h * pgate

    iota = lax.broadcasted_iota(jnp.int32, (B, N), 0)
    m = (iota == bat_row).astype(jnp.float32)
    cnt = jnp.sum(m, axis=1, keepdims=True)
    gap = jnp.dot(m, hp, preferred_element_type=jnp.float32) / jnp.maximum(cnt, 1.0)

    NEG = -3.4e38
    sc = hp
    k = 1
    while k < N:
        shifted = jnp.concatenate(
            [jnp.full((k, H), NEG, jnp.float32), sc[: N - k]], axis=0)
        bshift = jnp.concatenate(
            [jnp.full((k, 1), -1, jnp.int32), bat_col[: N - k]], axis=0)
        sc = jnp.maximum(sc, jnp.where(bshift == bat_col, shifted, NEG))
        k *= 2
    nxt_row = jnp.concatenate(
        [bat_row[:, 1:], jnp.full((1, 1), -1, jnp.int32)], axis=1)
    islast = (bat_row != nxt_row).astype(jnp.float32)
    gmax = jnp.dot(m * islast, sc, preferred_element_type=jnp.float32)
    gmax = jnp.where(cnt > 0.0, gmax, 0.0)
    return hp, kl, ref_, jnp.concatenate([gmax, gap], axis=1)


def _tc_mid(p, inv, bvec, svec, Wn, bat_col, bat_row):
    def body(p_ref, inv_ref, b_ref, s_ref, w_ref, bc_ref, br_ref,
             y_ref, x1_ref, kl_ref, ref_ref):
        hp, kl, ref_, xout = _post_layer(
            p_ref[0], p_ref[1], inv_ref[...], b_ref[...], s_ref[...],
            bc_ref[...], br_ref[...])
        x1_ref[...] = xout
        kl_ref[...] = kl.reshape(1, 1)
        ref_ref[...] = ref_.reshape(1, 1)
        y_ref[...] = (
            jnp.dot(hp, w_ref[...], preferred_element_type=jnp.float32)
            * inv_ref[...]
        )

    return pl.pallas_call(
        body,
        out_shape=(
            jax.ShapeDtypeStruct((N, H), jnp.float32),
            jax.ShapeDtypeStruct((B, 2 * H), jnp.float32),
            jax.ShapeDtypeStruct((1, 1), jnp.float32),
            jax.ShapeDtypeStruct((1, 1), jnp.float32),
        ),
    )(p, inv, bvec, svec, Wn, bat_col, bat_row)


def _tc_final(p, inv, bvec, svec, bat_col, bat_row, x1, x2, kl1, kl2, ref1, ref2,
              Wl1, bl1, Wl2, bl2, Wl3, bl3):
    def body(p_ref, inv_ref, b_ref, s_ref, bc_ref, br_ref, x1_ref, x2_ref,
             kl1_ref, kl2_ref, ref1_ref, ref2_ref,
             wl1_ref, bl1_ref, wl2_ref, bl2_ref, wl3_ref, bl3_ref,
             logits_ref, kl_ref, ref_ref):
        hp, kl3, ref3, x3 = _post_layer(
            p_ref[0], p_ref[1], inv_ref[...], b_ref[...], s_ref[...],
            bc_ref[...], br_ref[...])
        del hp
        kl_ref[...] = (kl1_ref[...] + kl2_ref[...] + kl3.reshape(1, 1))
        ref_ref[...] = (ref1_ref[...] + ref2_ref[...] + ref3.reshape(1, 1))

        g = (jnp.maximum(x1_ref[...], 0.0)
             + jnp.maximum(x2_ref[...], 0.0)
             + jnp.maximum(x3, 0.0))
        g = jnp.maximum(
            jnp.dot(g, wl1_ref[...], preferred_element_type=jnp.float32)
            + bl1_ref[...], 0.0)
        g = jnp.maximum(
            jnp.dot(g, wl2_ref[...], preferred_element_type=jnp.float32)
            + bl2_ref[...], 0.0)
        z = (jnp.dot(g, wl3_ref[...], preferred_element_type=jnp.float32)
             + bl3_ref[...])
        zm = jnp.max(z, axis=1, keepdims=True)
        ze = z - zm
        logits_ref[...] = ze - jnp.log(jnp.sum(jnp.exp(ze), axis=1, keepdims=True))

    return pl.pallas_call(
        body,
        out_shape=(
            jax.ShapeDtypeStruct((B, C), jnp.float32),
            jax.ShapeDtypeStruct((1, 1), jnp.float32),
            jax.ShapeDtypeStruct((1, 1), jnp.float32),
        ),
    )(p, inv, bvec, svec, bat_col, bat_row, x1, x2, kl1, kl2, ref1, ref2,
      Wl1, bl1, Wl2, bl2, Wl3, bl3)


def kernel(x, edge_index, batch, W1, b1, W2, b2, W3, b3, s1, s2, s3,
           Wl1, bl1, Wl2, bl2, Wl3, bl3):
    src = edge_index[0]
    dst = edge_index[1]
    bat_col = batch.reshape(N, 1)
    bat_row = batch.reshape(1, N)
    zeros_nh = jnp.zeros((N, H), jnp.float32)
    zeros_n = jnp.zeros((N,), jnp.float32)
    ones_ch = jnp.ones((CH,), jnp.float32)

    degp = _sc_degree(dst, ones_ch, zeros_n).reshape(NC, N, 1)
    y1, inv = _tc_pre(x, W1, degp)
    p1 = _sc_spmm(y1, src, dst, zeros_nh)
    y2, x1, kl1, ref1 = _tc_mid(
        p1, inv, b1.reshape(1, H), s1.reshape(H, 1), W2, bat_col, bat_row)
    p2 = _sc_spmm(y2, src, dst, zeros_nh)
    y3, x2, kl2, ref2 = _tc_mid(
        p2, inv, b2.reshape(1, H), s2.reshape(H, 1), W3, bat_col, bat_row)
    p3 = _sc_spmm(y3, src, dst, zeros_nh)
    logits, kl, refo = _tc_final(
        p3, inv, b3.reshape(1, H), s3.reshape(H, 1), bat_col, bat_row,
        x1, x2, kl1, kl2, ref1, ref2,
        Wl1, bl1.reshape(1, H), Wl2, bl2.reshape(1, H // 2),
        Wl3, bl3.reshape(1, C))
    return (logits, kl.reshape(()), refo.reshape(()))


# speedup vs baseline: 18.3497x; 1.4387x over previous
import functools

import jax
import jax.numpy as jnp
from jax import lax
from jax.experimental import pallas as pl
from jax.experimental.pallas import tpu as pltpu
from jax.experimental.pallas import tpu_sc as plsc

N = 10000
E = 320000
H = 128
B = 64
C = 10

NC = 2
NS = 16
NW = NC * NS
EPW = E // NW
CH = 80
NCHUNK = EPW // CH
RPT = 624
TAIL = N - NS * RPT
TAIL_OFF = NS * RPT


def _sc_degree(dst, ones_ch, zeros_n):
    mesh = plsc.VectorSubcoreMesh(core_axis_name="c", subcore_axis_name="s")

    @functools.partial(
        pl.kernel,
        out_type=jax.ShapeDtypeStruct((NC * N,), jnp.float32),
        mesh=mesh,
    scratch_types=[
            pltpu.VMEM((EPW,), jnp.int32),
            pltpu.VMEM((CH,), jnp.float32),
            pltpu.VMEM((RPT + TAIL,), jnp.float32),
            pltpu.VMEM_SHARED((N,), jnp.float32),
        ],
    )
    def k(dst_hbm, ones_hbm, zero_hbm, out_hbm, di_big, ones_v, stage_v, acc_sh):
        cid = lax.axis_index("c")
        sid = lax.axis_index("s")
        wid = sid * NC + cid

        pltpu.sync_copy(ones_hbm, ones_v)
        pltpu.sync_copy(dst_hbm.at[pl.ds(wid * EPW, EPW)], di_big)
        pltpu.sync_copy(zero_hbm.at[pl.ds(0, RPT + TAIL)], stage_v)
        pltpu.sync_copy(
            stage_v.at[pl.ds(0, RPT)], acc_sh.at[pl.ds(sid * RPT, RPT)]
        )

        @pl.when(sid == NS - 1)
        def _():
            pltpu.sync_copy(
                stage_v.at[pl.ds(RPT, TAIL)], acc_sh.at[pl.ds(TAIL_OFF, TAIL)]
            )

        plsc.subcore_barrier()

        def chunk(i, carry):
            pltpu.sync_copy(
                ones_v, acc_sh.at[di_big.at[pl.ds(i * CH, CH)]], add=True)
            return carry

        lax.fori_loop(0, NCHUNK, chunk, 0)
        plsc.subcore_barrier()
        pltpu.sync_copy(acc_sh.at[pl.ds(sid * RPT, RPT)], stage_v.at[pl.ds(0, RPT)])

        @pl.when(sid == NS - 1)
        def _():
            pltpu.sync_copy(
                acc_sh.at[pl.ds(TAIL_OFF, TAIL)], stage_v.at[pl.ds(RPT, TAIL)]
            )

        pltpu.sync_copy(
            stage_v.at[pl.ds(0, RPT)], out_hbm.at[pl.ds(cid * N + sid * RPT, RPT)]
        )

        @pl.when(sid == NS - 1)
        def _():
            pltpu.sync_copy(
                stage_v.at[pl.ds(RPT, TAIL)],
                out_hbm.at[pl.ds(cid * N + TAIL_OFF, TAIL)],
            )

    return k(dst, ones_ch, zeros_n)


def _sc_spmm(y, src, dst, zeros_nh):
    mesh = plsc.VectorSubcoreMesh(core_axis_name="c", subcore_axis_name="s")

    @functools.partial(
        pl.kernel,
        out_type=jax.ShapeDtypeStruct((NC, N, H), jnp.float32),
        mesh=mesh,
        scratch_types=[
            pltpu.VMEM((EPW,), jnp.int32),
            pltpu.VMEM((EPW,), jnp.int32),
            pltpu.VMEM((CH, H), jnp.float32),
            pltpu.VMEM((CH, H), jnp.float32),
            pltpu.VMEM_SHARED((N, H), jnp.float32),
            pltpu.SemaphoreType.DMA,
        ],
    )
    def k(y_hbm, src_hbm, dst_hbm, zero_hbm, out_hbm,
          si_big, di_big, rows0, rows1, acc_sh, gsem):
        cid = lax.axis_index("c")
        sid = lax.axis_index("s")
        wid = sid * NC + cid

        pltpu.sync_copy(
            zero_hbm.at[pl.ds(sid * RPT, RPT)], acc_sh.at[pl.ds(sid * RPT, RPT)]
        )

        @pl.when(sid == NS - 1)
        def _():
            pltpu.sync_copy(
                zero_hbm.at[pl.ds(TAIL_OFF, TAIL)], acc_sh.at[pl.ds(TAIL_OFF, TAIL)]
            )

        base0 = wid * EPW
        pltpu.sync_copy(src_hbm.at[pl.ds(base0, EPW)], si_big)
        pltpu.sync_copy(dst_hbm.at[pl.ds(base0, EPW)], di_big)
        plsc.subcore_barrier()

        pltpu.async_copy(y_hbm.at[si_big.at[pl.ds(0, CH)]], rows0, gsem)
        pltpu.async_copy(y_hbm.at[si_big.at[pl.ds(CH, CH)]], rows1, gsem)

        def pair(j, carry):
            c = 2 * j * CH
            pltpu.make_async_copy(y_hbm.at[si_big.at[pl.ds(0, CH)]], rows0,
                                  gsem).wait()
            pltpu.sync_copy(rows0, acc_sh.at[di_big.at[pl.ds(c, CH)]], add=True)
            pltpu.async_copy(y_hbm.at[si_big.at[pl.ds(c + 2 * CH, CH)]], rows0,
                             gsem)
            pltpu.make_async_copy(y_hbm.at[si_big.at[pl.ds(0, CH)]], rows1,
                                  gsem).wait()
            pltpu.sync_copy(rows1, acc_sh.at[di_big.at[pl.ds(c + CH, CH)]],
                            add=True)
            pltpu.async_copy(y_hbm.at[si_big.at[pl.ds(c + 3 * CH, CH)]], rows1,
                             gsem)
            return carry

        lax.fori_loop(0, (NCHUNK - 1) // 2 - 1, pair, 0)
        c = (NCHUNK - 3) * CH
        pltpu.make_async_copy(y_hbm.at[si_big.at[pl.ds(0, CH)]], rows0,
                              gsem).wait()
        pltpu.sync_copy(rows0, acc_sh.at[di_big.at[pl.ds(c, CH)]], add=True)
        pltpu.async_copy(y_hbm.at[si_big.at[pl.ds(c + 2 * CH, CH)]], rows0, gsem)
        pltpu.make_async_copy(y_hbm.at[si_big.at[pl.ds(0, CH)]], rows1,
                              gsem).wait()
        pltpu.sync_copy(rows1, acc_sh.at[di_big.at[pl.ds(c + CH, CH)]], add=True)
        pltpu.make_async_copy(y_hbm.at[si_big.at[pl.ds(0, CH)]], rows0,
                              gsem).wait()
        pltpu.sync_copy(rows0, acc_sh.at[di_big.at[pl.ds(c + 2 * CH, CH)]],
                        add=True)
        plsc.subcore_barrier()
        pltpu.sync_copy(
            acc_sh.at[pl.ds(sid * RPT, RPT)], out_hbm.at[cid, pl.ds(sid * RPT, RPT)]
        )

        @pl.when(sid == NS - 1)
        def _():
            pltpu.sync_copy(
                acc_sh.at[pl.ds(TAIL_OFF, TAIL)],
                out_hbm.at[cid, pl.ds(TAIL_OFF, TAIL)],
            )

    return k(y, src, dst, zeros_nh)


def _tc_pre(x, W1, degp):
    def body(x_ref, w_ref, degp_ref, y_ref, inv_ref):
        deg = degp_ref[0] + degp_ref[1]
        inv = lax.rsqrt(jnp.maximum(deg, 1.0))
        h = jnp.dot(x_ref[...], w_ref[...], preferred_element_type=jnp.float32)
        y_ref[...] = h * inv
        inv_ref[...] = inv

    return pl.pallas_call(
        body,
        out_shape=(
            jax.ShapeDtypeStruct((N, H), jnp.float32),
            jax.ShapeDtypeStruct((N, 1), jnp.float32),
        ),
    )(x, W1, degp)


def _post_layer(p0, p1, inv, bvec, svec, bat_col, bat_row):
    agg = (p0 + p1) * inv + bvec
    h = jnp.maximum(agg, 0.0)
    logit = jnp.dot(h, svec, preferred_element_type=jnp.float32)
    pgate = 1.0 / (1.0 + jnp.exp(-logit))
    pc = jnp.clip(pgate, 1e-6, 1.0 - 1e-6)
    kl = jnp.mean(pc * jnp.log(2.0 * pc) + (1.0 - pc) * jnp.log(2.0 * (1.0 - pc)))
    ref_ = jnp.mean((pc - 0.5) ** 2)
    hp = h * pgate

    iota = lax.broadcasted_iota(jnp.int32, (B, N), 0)
    m = (iota == bat_row).astype(jnp.float32)
    cnt = jnp.sum(m, axis=1, keepdims=True)
    gap = jnp.dot(m, hp, preferred_element_type=jnp.float32) / jnp.maximum(cnt, 1.0)

    NEG = -3.4e38
    sc = hp
    k = 1
    while k < N:
        shifted = jnp.concatenate(
            [jnp.full((k, H), NEG, jnp.float32), sc[: N - k]], axis=0)
        bshift = jnp.concatenate(
            [jnp.full((k, 1), -1, jnp.int32), bat_col[: N - k]], axis=0)
        sc = jnp.maximum(sc, jnp.where(bshift == bat_col, shifted, NEG))
        k *= 2
    nxt_row = jnp.concatenate(
        [bat_row[:, 1:], jnp.full((1, 1), -1, jnp.int32)], axis=1)
    islast = (bat_row != nxt_row).astype(jnp.float32)
    gmax = jnp.dot(m * islast, sc, preferred_element_type=jnp.float32)
    gmax = jnp.where(cnt > 0.0, gmax, 0.0)
    return hp, kl, ref_, jnp.concatenate([gmax, gap], axis=1)


def _tc_mid(p, inv, bvec, svec, Wn, bat_col, bat_row):
    def body(p_ref, inv_ref, b_ref, s_ref, w_ref, bc_ref, br_ref,
             y_ref, x1_ref, kl_ref, ref_ref):
        hp, kl, ref_, xout = _post_layer(
            p_ref[0], p_ref[1], inv_ref[...], b_ref[...], s_ref[...],
            bc_ref[...], br_ref[...])
        x1_ref[...] = xout
        kl_ref[...] = kl.reshape(1, 1)
        ref_ref[...] = ref_.reshape(1, 1)
        y_ref[...] = (
            jnp.dot(hp, w_ref[...], preferred_element_type=jnp.float32)
            * inv_ref[...]
        )

    return pl.pallas_call(
        body,
        out_shape=(
            jax.ShapeDtypeStruct((N, H), jnp.float32),
            jax.ShapeDtypeStruct((B, 2 * H), jnp.float32),
            jax.ShapeDtypeStruct((1, 1), jnp.float32),
            jax.ShapeDtypeStruct((1, 1), jnp.float32),
        ),
    )(p, inv, bvec, svec, Wn, bat_col, bat_row)


def _tc_final(p, inv, bvec, svec, bat_col, bat_row, x1, x2, kl1, kl2, ref1, ref2,
              Wl1, bl1, Wl2, bl2, Wl3, bl3):
    def body(p_ref, inv_ref, b_ref, s_ref, bc_ref, br_ref, x1_ref, x2_ref,
             kl1_ref, kl2_ref, ref1_ref, ref2_ref,
             wl1_ref, bl1_ref, wl2_ref, bl2_ref, wl3_ref, bl3_ref,
             logits_ref, kl_ref, ref_ref):
        hp, kl3, ref3, x3 = _post_layer(
            p_ref[0], p_ref[1], inv_ref[...], b_ref[...], s_ref[...],
            bc_ref[...], br_ref[...])
        del hp
        kl_ref[...] = (kl1_ref[...] + kl2_ref[...] + kl3.reshape(1, 1))
        ref_ref[...] = (ref1_ref[...] + ref2_ref[...] + ref3.reshape(1, 1))

        g = (jnp.maximum(x1_ref[...], 0.0)
             + jnp.maximum(x2_ref[...], 0.0)
             + jnp.maximum(x3, 0.0))
        g = jnp.maximum(
            jnp.dot(g, wl1_ref[...], preferred_element_type=jnp.float32)
            + bl1_ref[...], 0.0)
        g = jnp.maximum(
            jnp.dot(g, wl2_ref[...], preferred_element_type=jnp.float32)
            + bl2_ref[...], 0.0)
        z = (jnp.dot(g, wl3_ref[...], preferred_element_type=jnp.float32)
             + bl3_ref[...])
        zm = jnp.max(z, axis=1, keepdims=True)
        ze = z - zm
        logits_ref[...] = ze - jnp.log(jnp.sum(jnp.exp(ze), axis=1, keepdims=True))

    return pl.pallas_call(
        body,
        out_shape=(
            jax.ShapeDtypeStruct((B, C), jnp.float32),
            jax.ShapeDtypeStruct((1, 1), jnp.float32),
            jax.ShapeDtypeStruct((1, 1), jnp.float32),
        ),
    )(p, inv, bvec, svec, bat_col, bat_row, x1, x2, kl1, kl2, ref1, ref2,
      Wl1, bl1, Wl2, bl2, Wl3, bl3)


def kernel(x, edge_index, batch, W1, b1, W2, b2, W3, b3, s1, s2, s3,
           Wl1, bl1, Wl2, bl2, Wl3, bl3):
    src = edge_index[0]
    dst = edge_index[1]
    bat_col = batch.reshape(N, 1)
    bat_row = batch.reshape(1, N)
    zeros_nh = jnp.zeros((N, H), jnp.float32)
    zeros_n = jnp.zeros((N,), jnp.float32)
    ones_ch = jnp.ones((CH,), jnp.float32)

    degp = _sc_degree(dst, ones_ch, zeros_n).reshape(NC, N, 1)
    y1, inv = _tc_pre(x, W1, degp)
    p1 = _sc_spmm(y1, src, dst, zeros_nh)
    y2, x1, kl1, ref1 = _tc_mid(
        p1, inv, b1.reshape(1, H), s1.reshape(H, 1), W2, bat_col, bat_row)
    p2 = _sc_spmm(y2, src, dst, zeros_nh)
    y3, x2, kl2, ref2 = _tc_mid(
        p2, inv, b2.reshape(1, H), s2.reshape(H, 1), W3, bat_col, bat_row)
    p3 = _sc_spmm(y3, src, dst, zeros_nh)
    logits, kl, refo = _tc_final(
        p3, inv, b3.reshape(1, H), s3.reshape(H, 1), bat_col, bat_row,
        x1, x2, kl1, kl2, ref1, ref2,
        Wl1, bl1.reshape(1, H), Wl2, bl2.reshape(1, H // 2),
        Wl3, bl3.reshape(1, C))
    return (logits, kl.reshape(()), refo.reshape(()))


# speedup vs baseline: 20.8142x; 1.1343x over previous
import functools

import jax
import jax.numpy as jnp
from jax import lax
from jax.experimental import pallas as pl
from jax.experimental.pallas import tpu as pltpu
from jax.experimental.pallas import tpu_sc as plsc

N = 10000
E = 320000
H = 128
B = 64
C = 10

NC = 2
NS = 16
NW = NC * NS
EPW = E // NW
CH = 80
NCHUNK = EPW // CH
RPT = 624
TAIL = N - NS * RPT
TAIL_OFF = NS * RPT


def _sc_degree(dst, ones_ch, zeros_n):
    mesh = plsc.VectorSubcoreMesh(core_axis_name="c", subcore_axis_name="s")

    @functools.partial(
        pl.kernel,
        out_type=jax.ShapeDtypeStruct((NC * N,), jnp.float32),
        mesh=mesh,
    scratch_types=[
            pltpu.VMEM((EPW,), jnp.int32),
            pltpu.VMEM((CH,), jnp.float32),
            pltpu.VMEM((RPT + TAIL,), jnp.float32),
            pltpu.VMEM_SHARED((N,), jnp.float32),
        ],
    )
    def k(dst_hbm, ones_hbm, zero_hbm, out_hbm, di_big, ones_v, stage_v, acc_sh):
        cid = lax.axis_index("c")
        sid = lax.axis_index("s")
        wid = sid * NC + cid

        pltpu.sync_copy(ones_hbm, ones_v)
        pltpu.sync_copy(dst_hbm.at[pl.ds(wid * EPW, EPW)], di_big)
        pltpu.sync_copy(zero_hbm.at[pl.ds(0, RPT + TAIL)], stage_v)
        pltpu.sync_copy(
            stage_v.at[pl.ds(0, RPT)], acc_sh.at[pl.ds(sid * RPT, RPT)]
        )

        @pl.when(sid == NS - 1)
        def _():
            pltpu.sync_copy(
                stage_v.at[pl.ds(RPT, TAIL)], acc_sh.at[pl.ds(TAIL_OFF, TAIL)]
            )

        plsc.subcore_barrier()

        def chunk(i, carry):
            pltpu.sync_copy(
                ones_v, acc_sh.at[di_big.at[pl.ds(i * CH, CH)]], add=True)
            return carry

        lax.fori_loop(0, NCHUNK, chunk, 0)
        plsc.subcore_barrier()
        pltpu.sync_copy(acc_sh.at[pl.ds(sid * RPT, RPT)], stage_v.at[pl.ds(0, RPT)])

        @pl.when(sid == NS - 1)
        def _():
            pltpu.sync_copy(
                acc_sh.at[pl.ds(TAIL_OFF, TAIL)], stage_v.at[pl.ds(RPT, TAIL)]
            )

        pltpu.sync_copy(
            stage_v.at[pl.ds(0, RPT)], out_hbm.at[pl.ds(cid * N + sid * RPT, RPT)]
        )

        @pl.when(sid == NS - 1)
        def _():
            pltpu.sync_copy(
                stage_v.at[pl.ds(RPT, TAIL)],
                out_hbm.at[pl.ds(cid * N + TAIL_OFF, TAIL)],
            )

    return k(dst, ones_ch, zeros_n)


def _sc_spmm(y, src, dst, zeros_nh):
    mesh = plsc.VectorSubcoreMesh(core_axis_name="c", subcore_axis_name="s")

    @functools.partial(
        pl.kernel,
        out_type=jax.ShapeDtypeStruct((NC, N, H), jnp.float32),
        mesh=mesh,
        scratch_types=[
            pltpu.VMEM((EPW,), jnp.int32),
            pltpu.VMEM((EPW,), jnp.int32),
            pltpu.VMEM((CH, H), jnp.float32),
            pltpu.VMEM((CH, H), jnp.float32),
            pltpu.VMEM((CH, H), jnp.float32),
            pltpu.VMEM_SHARED((N, H), jnp.float32),
            pltpu.SemaphoreType.DMA,
            pltpu.SemaphoreType.DMA,
            pltpu.SemaphoreType.DMA,
        ],
    )
    def k(y_hbm, src_hbm, dst_hbm, zero_hbm, out_hbm,
          si_big, di_big, rows0, rows1, rows2, acc_sh, gsem0, gsem1, gsem2):
        cid = lax.axis_index("c")
        sid = lax.axis_index("s")
        wid = sid * NC + cid

        pltpu.sync_copy(
            zero_hbm.at[pl.ds(sid * RPT, RPT)], acc_sh.at[pl.ds(sid * RPT, RPT)]
        )

        @pl.when(sid == NS - 1)
        def _():
            pltpu.sync_copy(
                zero_hbm.at[pl.ds(TAIL_OFF, TAIL)], acc_sh.at[pl.ds(TAIL_OFF, TAIL)]
            )

        base0 = wid * EPW
        pltpu.sync_copy(src_hbm.at[pl.ds(base0, EPW)], si_big)
        pltpu.sync_copy(dst_hbm.at[pl.ds(base0, EPW)], di_big)
        plsc.subcore_barrier()

        sems = {id(rows0): gsem0, id(rows1): gsem1, id(rows2): gsem2}

        def gat(c, rows):
            pltpu.async_copy(y_hbm.at[si_big.at[pl.ds(c * CH, CH)]], rows,
                             sems[id(rows)])

        def wt(rows):
            pltpu.make_async_copy(y_hbm.at[si_big.at[pl.ds(0, CH)]], rows,
                                  sems[id(rows)]).wait()

        def sca(c, rows):
            pltpu.sync_copy(rows, acc_sh.at[di_big.at[pl.ds(c * CH, CH)]],
                            add=True)

        gat(0, rows0)
        gat(1, rows1)
        gat(2, rows2)

        def triple(j, carry):
            c = 3 * j
            wt(rows0); sca(c, rows0); gat(c + 3, rows0)
            wt(rows1); sca(c + 1, rows1); gat(c + 4, rows1)
            wt(rows2); sca(c + 2, rows2); gat(c + 5, rows2)
            return carry

        lax.fori_loop(0, (NCHUNK - 5) // 3, triple, 0)
        c = NCHUNK - 5
        wt(rows0); sca(c, rows0); gat(c + 3, rows0)
        wt(rows1); sca(c + 1, rows1); gat(c + 4, rows1)
        wt(rows2); sca(c + 2, rows2)
        wt(rows0); sca(c + 3, rows0)
        wt(rows1); sca(c + 4, rows1)
        plsc.subcore_barrier()
        pltpu.sync_copy(
            acc_sh.at[pl.ds(sid * RPT, RPT)], out_hbm.at[cid, pl.ds(sid * RPT, RPT)]
        )

        @pl.when(sid == NS - 1)
        def _():
            pltpu.sync_copy(
                acc_sh.at[pl.ds(TAIL_OFF, TAIL)],
                out_hbm.at[cid, pl.ds(TAIL_OFF, TAIL)],
            )

    return k(y, src, dst, zeros_nh)


def _tc_pre(x, W1, degp):
    def body(x_ref, w_ref, degp_ref, y_ref, inv_ref):
        deg = degp_ref[0] + degp_ref[1]
        inv = lax.rsqrt(jnp.maximum(deg, 1.0))
        h = jnp.dot(x_ref[...], w_ref[...], preferred_element_type=jnp.float32)
        y_ref[...] = h * inv
        inv_ref[...] = inv

    return pl.pallas_call(
        body,
        out_shape=(
            jax.ShapeDtypeStruct((N, H), jnp.float32),
            jax.ShapeDtypeStruct((N, 1), jnp.float32),
        ),
    )(x, W1, degp)


def _post_layer(p0, p1, inv, bvec, svec, bat_col, bat_row):
    agg = (p0 + p1) * inv + bvec
    h = jnp.maximum(agg, 0.0)
    logit = jnp.dot(h, svec, preferred_element_type=jnp.float32)
    pgate = 1.0 / (1.0 + jnp.exp(-logit))
    pc = jnp.clip(pgate, 1e-6, 1.0 - 1e-6)
    kl = jnp.mean(pc * jnp.log(2.0 * pc) + (1.0 - pc) * jnp.log(2.0 * (1.0 - pc)))
    ref_ = jnp.mean((pc - 0.5) ** 2)
    hp = h * pgate

    iota = lax.broadcasted_iota(jnp.int32, (B, N), 0)
    m = (iota == bat_row).astype(jnp.float32)
    cnt = jnp.sum(m, axis=1, keepdims=True)
    gap = jnp.dot(m, hp, preferred_element_type=jnp.float32) / jnp.maximum(cnt, 1.0)

    NEG = -3.4e38
    sc = hp
    k = 1
    while k < N:
        shifted = jnp.concatenate(
            [jnp.full((k, H), NEG, jnp.float32), sc[: N - k]], axis=0)
        bshift = jnp.concatenate(
            [jnp.full((k, 1), -1, jnp.int32), bat_col[: N - k]], axis=0)
        sc = jnp.maximum(sc, jnp.where(bshift == bat_col, shifted, NEG))
        k *= 2
    nxt_row = jnp.concatenate(
        [bat_row[:, 1:], jnp.full((1, 1), -1, jnp.int32)], axis=1)
    islast = (bat_row != nxt_row).astype(jnp.float32)
    gmax = jnp.dot(m * islast, sc, preferred_element_type=jnp.float32)
    gmax = jnp.where(cnt > 0.0, gmax, 0.0)
    return hp, kl, ref_, jnp.concatenate([gmax, gap], axis=1)


def _tc_mid(p, inv, bvec, svec, Wn, bat_col, bat_row):
    def body(p_ref, inv_ref, b_ref, s_ref, w_ref, bc_ref, br_ref,
             y_ref, x1_ref, kl_ref, ref_ref):
        hp, kl, ref_, xout = _post_layer(
            p_ref[0], p_ref[1], inv_ref[...], b_ref[...], s_ref[...],
            bc_ref[...], br_ref[...])
        x1_ref[...] = xout
        kl_ref[...] = kl.reshape(1, 1)
        ref_ref[...] = ref_.reshape(1, 1)
        y_ref[...] = (
            jnp.dot(hp, w_ref[...], preferred_element_type=jnp.float32)
            * inv_ref[...]
        )

    return pl.pallas_call(
        body,
        out_shape=(
            jax.ShapeDtypeStruct((N, H), jnp.float32),
            jax.ShapeDtypeStruct((B, 2 * H), jnp.float32),
            jax.ShapeDtypeStruct((1, 1), jnp.float32),
            jax.ShapeDtypeStruct((1, 1), jnp.float32),
        ),
    )(p, inv, bvec, svec, Wn, bat_col, bat_row)


def _tc_final(p, inv, bvec, svec, bat_col, bat_row, x1, x2, kl1, kl2, ref1, ref2,
              Wl1, bl1, Wl2, bl2, Wl3, bl3):
    def body(p_ref, inv_ref, b_ref, s_ref, bc_ref, br_ref, x1_ref, x2_ref,
             kl1_ref, kl2_ref, ref1_ref, ref2_ref,
             wl1_ref, bl1_ref, wl2_ref, bl2_ref, wl3_ref, bl3_ref,
             logits_ref, kl_ref, ref_ref):
        hp, kl3, ref3, x3 = _post_layer(
            p_ref[0], p_ref[1], inv_ref[...], b_ref[...], s_ref[...],
            bc_ref[...], br_ref[...])
        del hp
        kl_ref[...] = (kl1_ref[...] + kl2_ref[...] + kl3.reshape(1, 1))
        ref_ref[...] = (ref1_ref[...] + ref2_ref[...] + ref3.reshape(1, 1))

        g = (jnp.maximum(x1_ref[...], 0.0)
             + jnp.maximum(x2_ref[...], 0.0)
             + jnp.maximum(x3, 0.0))
        g = jnp.maximum(
            jnp.dot(g, wl1_ref[...], preferred_element_type=jnp.float32)
            + bl1_ref[...], 0.0)
        g = jnp.maximum(
            jnp.dot(g, wl2_ref[...], preferred_element_type=jnp.float32)
            + bl2_ref[...], 0.0)
        z = (jnp.dot(g, wl3_ref[...], preferred_element_type=jnp.float32)
             + bl3_ref[...])
        zm = jnp.max(z, axis=1, keepdims=True)
        ze = z - zm
        logits_ref[...] = ze - jnp.log(jnp.sum(jnp.exp(ze), axis=1, keepdims=True))

    return pl.pallas_call(
        body,
        out_shape=(
            jax.ShapeDtypeStruct((B, C), jnp.float32),
            jax.ShapeDtypeStruct((1, 1), jnp.float32),
            jax.ShapeDtypeStruct((1, 1), jnp.float32),
        ),
    )(p, inv, bvec, svec, bat_col, bat_row, x1, x2, kl1, kl2, ref1, ref2,
      Wl1, bl1, Wl2, bl2, Wl3, bl3)


def kernel(x, edge_index, batch, W1, b1, W2, b2, W3, b3, s1, s2, s3,
           Wl1, bl1, Wl2, bl2, Wl3, bl3):
    src = edge_index[0]
    dst = edge_index[1]
    bat_col = batch.reshape(N, 1)
    bat_row = batch.reshape(1, N)
    zeros_nh = jnp.zeros((N, H), jnp.float32)
    zeros_n = jnp.zeros((N,), jnp.float32)
    ones_ch = jnp.ones((CH,), jnp.float32)

    degp = _sc_degree(dst, ones_ch, zeros_n).reshape(NC, N, 1)
    y1, inv = _tc_pre(x, W1, degp)
    p1 = _sc_spmm(y1, src, dst, zeros_nh)
    y2, x1, kl1, ref1 = _tc_mid(
        p1, inv, b1.reshape(1, H), s1.reshape(H, 1), W2, bat_col, bat_row)
    p2 = _sc_spmm(y2, src, dst, zeros_nh)
    y3, x2, kl2, ref2 = _tc_mid(
        p2, inv, b2.reshape(1, H), s2.reshape(H, 1), W3, bat_col, bat_row)
    p3 = _sc_spmm(y3, src, dst, zeros_nh)
    logits, kl, refo = _tc_final(
        p3, inv, b3.reshape(1, H), s3.reshape(H, 1), bat_col, bat_row,
        x1, x2, kl1, kl2, ref1, ref2,
        Wl1, bl1.reshape(1, H), Wl2, bl2.reshape(1, H // 2),
        Wl3, bl3.reshape(1, C))
    return (logits, kl.reshape(()), refo.reshape(()))
